# Initial kernel scaffold; baseline (speedup 1.0000x reference)
import jax
import jax.numpy as jnp
from jax.experimental import pallas as pl


def kernel(p_feat, v_feat, r_feat, v2p_ind, r2p_ind, Wp, bp, Wv, bv, Wr, br, ap, av, ar):
    raise NotImplementedError("write your pallas kernel here")


# speedup vs baseline: 22.5293x; 22.5293x over previous
import functools

import jax
import jax.numpy as jnp
from jax import lax
from jax.experimental import pallas as pl
from jax.experimental.pallas import tpu as pltpu
from jax.experimental.pallas import tpu_sc as plsc

_NC = 2
_NS = 16
_NW = _NC * _NS


def _score_grid_body(f_ref, w_ref, a_ref, b_ref, s_ref):
    g = jnp.dot(a_ref[...], w_ref[...], preferred_element_type=jnp.float32)
    c0 = jnp.sum(a_ref[...] * b_ref[...])
    s_ref[0] = jnp.dot(g, f_ref[0], preferred_element_type=jnp.float32) + c0


def _score_grid(feat, W, a, b, T):
    Bn, C, G = feat.shape
    CE = W.shape[0]
    return pl.pallas_call(
        _score_grid_body,
        grid=(Bn, G // T),
        in_specs=[
            pl.BlockSpec((1, C, T), lambda bi, i: (bi, 0, i)),
            pl.BlockSpec((CE, C), lambda bi, i: (0, 0)),
            pl.BlockSpec((1, CE), lambda bi, i: (0, 0)),
            pl.BlockSpec((1, CE), lambda bi, i: (0, 0)),
        ],
        out_specs=pl.BlockSpec((1, 1, T), lambda bi, i: (bi, 0, i)),
        out_shape=jax.ShapeDtypeStruct((Bn, 1, G), jnp.float32),
    )(feat, W, a.reshape(1, CE), b.reshape(1, CE))


def _point_body(p_ref, sv_ref, sr_ref, w_ref, b_ref, a_ref,
                xp_ref, av_ref, ar_ref):
    emb = jnp.dot(w_ref[...], p_ref[0],
                  preferred_element_type=jnp.float32) + b_ref[...]
    sp = jnp.sum(emb * a_ref[...], axis=0, keepdims=True)
    sv = sv_ref[0]
    sr = sr_ref[0]
    m = jnp.maximum(sp, jnp.maximum(sv, sr))
    ep = jnp.exp(sp - m)
    ev = jnp.exp(sv - m)
    er = jnp.exp(sr - m)
    inv = 1.0 / (ep + ev + er)
    xp_ref[0] = emb * (ep * inv)
    av_ref[0] = ev * inv
    ar_ref[0] = er * inv


def _point_kernel(p_feat, sv_pt, sr_pt, Wp, bp, ap, T):
    Bn, C, N = p_feat.shape
    CE = Wp.shape[0]
    nt = pl.cdiv(N, T)
    return pl.pallas_call(
        _point_body,
        grid=(Bn, nt),
        in_specs=[
            pl.BlockSpec((1, C, T), lambda bi, i: (bi, 0, i)),
            pl.BlockSpec((1, 1, T), lambda bi, i: (bi, 0, i)),
            pl.BlockSpec((1, 1, T), lambda bi, i: (bi, 0, i)),
            pl.BlockSpec((CE, C), lambda bi, i: (0, 0)),
            pl.BlockSpec((CE, 1), lambda bi, i: (0, 0)),
            pl.BlockSpec((CE, 1), lambda bi, i: (0, 0)),
        ],
        out_specs=[
            pl.BlockSpec((1, CE, T), lambda bi, i: (bi, 0, i)),
            pl.BlockSpec((1, 1, T), lambda bi, i: (bi, 0, i)),
            pl.BlockSpec((1, 1, T), lambda bi, i: (bi, 0, i)),
        ],
        out_shape=[
            jax.ShapeDtypeStruct((Bn, CE, N), jnp.float32),
            jax.ShapeDtypeStruct((Bn, 1, N), jnp.float32),
            jax.ShapeDtypeStruct((Bn, 1, N), jnp.float32),
        ],
    )(p_feat, sv_pt, sr_pt, Wp, bp.reshape(CE, 1), ap.reshape(CE, 1))


def _map_body(f_ref, w_ref, b_ref, wt_ref, o_ref):
    emb = jnp.dot(w_ref[...], f_ref[0],
                  preferred_element_type=jnp.float32) + b_ref[...]
    wt = wt_ref[0, 0, 0, :] + wt_ref[1, 0, 0, :]
    o_ref[0] = emb * wt[None, :]


def _map_kernel(feat, W, b, wt, T):
    Bn, C, G = feat.shape
    CE = W.shape[0]
    return pl.pallas_call(
        _map_body,
        grid=(Bn, G // T),
        in_specs=[
            pl.BlockSpec((1, C, T), lambda bi, i: (bi, 0, i)),
            pl.BlockSpec((CE, C), lambda bi, i: (0, 0)),
            pl.BlockSpec((CE, 1), lambda bi, i: (0, 0)),
            pl.BlockSpec((2, 1, 1, T), lambda bi, i: (0, bi, 0, i)),
        ],
        out_specs=pl.BlockSpec((1, CE, T), lambda bi, i: (bi, 0, i)),
        out_shape=jax.ShapeDtypeStruct((Bn, CE, G), jnp.float32),
    )(feat, W, b.reshape(CE, 1), wt)


def _sc_gather(svg, srg, linv, linr):
    PT = linv.shape[0]
    CH = PT // _NW
    mesh = plsc.VectorSubcoreMesh(core_axis_name="c", subcore_axis_name="s")

    @functools.partial(
        pl.kernel, mesh=mesh,
        out_type=[jax.ShapeDtypeStruct((PT,), jnp.float32),
                  jax.ShapeDtypeStruct((PT,), jnp.float32)],
        scratch_types=[pltpu.VMEM((CH,), jnp.int32),
                       pltpu.VMEM((CH,), jnp.float32),
                       pltpu.SemaphoreType.DMA],
    )
    def gk(svg_h, srg_h, lv_h, lr_h, ov_h, or_h, idx_v, row_v, sem):
        wid = lax.axis_index("s") * _NC + lax.axis_index("c")
        base = wid * CH
        pltpu.sync_copy(lv_h.at[pl.ds(base, CH)], idx_v)
        pltpu.async_copy(svg_h.at[idx_v], row_v, sem).wait()
        pltpu.sync_copy(row_v, ov_h.at[pl.ds(base, CH)])
        pltpu.sync_copy(lr_h.at[pl.ds(base, CH)], idx_v)
        pltpu.async_copy(srg_h.at[idx_v], row_v, sem).wait()
        pltpu.sync_copy(row_v, or_h.at[pl.ds(base, CH)])

    return gk(svg, srg, linv, linr)


def _sc_scatter(avp, linv, arp, linr, zrow, GV, GR):
    PT = avp.shape[0]
    CH = PT // _NW
    SV = GV // _NS
    SR = GR // _NS
    mesh = plsc.VectorSubcoreMesh(core_axis_name="c", subcore_axis_name="s")

    @functools.partial(
        pl.kernel, mesh=mesh,
        out_type=[jax.ShapeDtypeStruct((_NC, GV), jnp.float32),
                  jax.ShapeDtypeStruct((_NC, GR), jnp.float32)],
        scratch_types=[pltpu.VMEM((CH,), jnp.int32),
                       pltpu.VMEM((CH,), jnp.float32),
                       pltpu.VMEM((SR,), jnp.float32),
                       pltpu.VMEM_SHARED((GV,), jnp.float32),
                       pltpu.VMEM_SHARED((GR,), jnp.float32),
                       pltpu.SemaphoreType.DMA],
    )
    def sk(av_h, lv_h, ar_h, lr_h, z_h, ov_h, or_h,
           idx_v, val_v, buf_v, accv, accr, sem):
        cid = lax.axis_index("c")
        sid = lax.axis_index("s")
        wid = sid * _NC + cid
        base = wid * CH
        pltpu.sync_copy(z_h.at[pl.ds(0, SV)], accv.at[pl.ds(sid * SV, SV)])
        pltpu.sync_copy(z_h, accr.at[pl.ds(sid * SR, SR)])
        plsc.subcore_barrier()
        pltpu.sync_copy(lv_h.at[pl.ds(base, CH)], idx_v)
        pltpu.sync_copy(av_h.at[pl.ds(base, CH)], val_v)
        pltpu.sync_copy(val_v, accv.at[idx_v], add=True)
        pltpu.sync_copy(lr_h.at[pl.ds(base, CH)], idx_v)
        pltpu.sync_copy(ar_h.at[pl.ds(base, CH)], val_v)
        pltpu.sync_copy(val_v, accr.at[idx_v], add=True)
        plsc.subcore_barrier()
        pltpu.sync_copy(accv.at[pl.ds(sid * SV, SV)], buf_v.at[pl.ds(0, SV)])
        pltpu.sync_copy(buf_v.at[pl.ds(0, SV)], ov_h.at[cid, pl.ds(sid * SV, SV)])
        pltpu.sync_copy(accr.at[pl.ds(sid * SR, SR)], buf_v)
        pltpu.sync_copy(buf_v, or_h.at[cid, pl.ds(sid * SR, SR)])

    return sk(avp, linv, arp, linr, zrow)


def kernel(p_feat, v_feat, r_feat, v2p_ind, r2p_ind,
           Wp, bp, Wv, bv, Wr, br, ap, av, ar):
    B, CP, N = p_feat.shape
    CV, HV, WVG = v_feat.shape[1:]
    CR, HR, WRG = r_feat.shape[1:]
    CE = Wp.shape[0]
    GV = HV * WVG
    GR = HR * WRG

    vf = v_feat.reshape(B, CV, GV)
    rf = r_feat.reshape(B, CR, GR)

    sv_grid = _score_grid(vf, Wv, av, bv, T=8192)
    sr_grid = _score_grid(rf, Wr, ar, br, T=8192)

    vi = v2p_ind.astype(jnp.int32)
    ri = r2p_ind.astype(jnp.int32)
    offv = (jnp.arange(B, dtype=jnp.int32) * GV)[:, None]
    offr = (jnp.arange(B, dtype=jnp.int32) * GR)[:, None]
    linv = (vi[..., 0] * WVG + vi[..., 1] + offv).reshape(-1)
    linr = (ri[..., 0] * WRG + ri[..., 1] + offr).reshape(-1)
    TOT = B * N
    PT = ((TOT + 255) // 256) * 256
    pad = PT - TOT
    ipad = jnp.zeros((pad,), jnp.int32)
    linv_p = jnp.concatenate([linv, ipad])
    linr_p = jnp.concatenate([linr, ipad])

    sv_f, sr_f = _sc_gather(sv_grid.reshape(-1), sr_grid.reshape(-1),
                            linv_p, linr_p)
    sv_pt = sv_f[:TOT].reshape(B, 1, N)
    sr_pt = sr_f[:TOT].reshape(B, 1, N)

    xp, alpha_v, alpha_r = _point_kernel(p_feat, sv_pt, sr_pt, Wp, bp, ap,
                                         T=2048)

    fpad = jnp.zeros((pad,), jnp.float32)
    av_p = jnp.concatenate([alpha_v.reshape(-1), fpad])
    ar_p = jnp.concatenate([alpha_r.reshape(-1), fpad])
    zrow = jnp.zeros((B * GR // _NS,), jnp.float32)
    wv2, wr2 = _sc_scatter(av_p, linv_p, ar_p, linr_p, zrow, B * GV, B * GR)

    xv = _map_kernel(vf, Wv, bv, wv2.reshape(_NC, B, 1, GV), T=8192)
    xr = _map_kernel(rf, Wr, br, wr2.reshape(_NC, B, 1, GR), T=8192)
    return (xp, xv.reshape(B, CE, HV, WVG), xr.reshape(B, CE, HR, WRG))


# speedup vs baseline: 22.6337x; 1.0046x over previous
import functools

import jax
import jax.numpy as jnp
from jax import lax
from jax.experimental import pallas as pl
from jax.experimental.pallas import tpu as pltpu
from jax.experimental.pallas import tpu_sc as plsc

_NC = 2
_NS = 16
_NW = _NC * _NS


def _split(total):
    ch = ((total + _NW - 1) // _NW + 7) // 8 * 8
    last = total - (_NW - 1) * ch
    assert last > 0 and last % 8 == 0 and ch % 8 == 0
    return ch, last


def _score_grid_body(f_ref, w_ref, a_ref, b_ref, s_ref):
    g = jnp.dot(a_ref[...], w_ref[...], preferred_element_type=jnp.float32)
    c0 = jnp.sum(a_ref[...] * b_ref[...])
    s_ref[0] = jnp.dot(g, f_ref[0], preferred_element_type=jnp.float32) + c0


def _score_grid(feat, W, a, b, T):
    Bn, C, G = feat.shape
    CE = W.shape[0]
    return pl.pallas_call(
        _score_grid_body,
        grid=(Bn, G // T),
        in_specs=[
            pl.BlockSpec((1, C, T), lambda bi, i: (bi, 0, i)),
            pl.BlockSpec((CE, C), lambda bi, i: (0, 0)),
            pl.BlockSpec((1, CE), lambda bi, i: (0, 0)),
            pl.BlockSpec((1, CE), lambda bi, i: (0, 0)),
        ],
        out_specs=pl.BlockSpec((1, 1, T), lambda bi, i: (bi, 0, i)),
        out_shape=jax.ShapeDtypeStruct((Bn, 1, G), jnp.float32),
        compiler_params=pltpu.CompilerParams(
            dimension_semantics=("parallel", "parallel")),
    )(feat, W, a.reshape(1, CE), b.reshape(1, CE))


def _point_body(p_ref, sv_ref, sr_ref, w_ref, b_ref, a_ref,
                xp_ref, av_ref, ar_ref):
    emb = jnp.dot(w_ref[...], p_ref[0],
                  preferred_element_type=jnp.float32) + b_ref[...]
    sp = jnp.sum(emb * a_ref[...], axis=0, keepdims=True)
    sv = sv_ref[0]
    sr = sr_ref[0]
    m = jnp.maximum(sp, jnp.maximum(sv, sr))
    ep = jnp.exp(sp - m)
    ev = jnp.exp(sv - m)
    er = jnp.exp(sr - m)
    inv = 1.0 / (ep + ev + er)
    xp_ref[0] = emb * (ep * inv)
    av_ref[0] = ev * inv
    ar_ref[0] = er * inv


def _point_kernel(p_feat, sv_pt, sr_pt, Wp, bp, ap, T):
    Bn, C, N = p_feat.shape
    CE = Wp.shape[0]
    nt = pl.cdiv(N, T)
    return pl.pallas_call(
        _point_body,
        grid=(Bn, nt),
        in_specs=[
            pl.BlockSpec((1, C, T), lambda bi, i: (bi, 0, i)),
            pl.BlockSpec((1, 1, T), lambda bi, i: (bi, 0, i)),
            pl.BlockSpec((1, 1, T), lambda bi, i: (bi, 0, i)),
            pl.BlockSpec((CE, C), lambda bi, i: (0, 0)),
            pl.BlockSpec((CE, 1), lambda bi, i: (0, 0)),
            pl.BlockSpec((CE, 1), lambda bi, i: (0, 0)),
        ],
        out_specs=[
            pl.BlockSpec((1, CE, T), lambda bi, i: (bi, 0, i)),
            pl.BlockSpec((1, 1, T), lambda bi, i: (bi, 0, i)),
            pl.BlockSpec((1, 1, T), lambda bi, i: (bi, 0, i)),
        ],
        out_shape=[
            jax.ShapeDtypeStruct((Bn, CE, N), jnp.float32),
            jax.ShapeDtypeStruct((Bn, 1, N), jnp.float32),
            jax.ShapeDtypeStruct((Bn, 1, N), jnp.float32),
        ],
        compiler_params=pltpu.CompilerParams(
            dimension_semantics=("parallel", "parallel")),
    )(p_feat, sv_pt, sr_pt, Wp, bp.reshape(CE, 1), ap.reshape(CE, 1))


def _map_body(f_ref, w_ref, b_ref, wt_ref, o_ref):
    emb = jnp.dot(w_ref[...], f_ref[0],
                  preferred_element_type=jnp.float32) + b_ref[...]
    wt = wt_ref[0, 0, 0, :] + wt_ref[1, 0, 0, :]
    o_ref[0] = emb * wt[None, :]


def _map_kernel(feat, W, b, wt, T):
    Bn, C, G = feat.shape
    CE = W.shape[0]
    return pl.pallas_call(
        _map_body,
        grid=(Bn, G // T),
        in_specs=[
            pl.BlockSpec((1, C, T), lambda bi, i: (bi, 0, i)),
            pl.BlockSpec((CE, C), lambda bi, i: (0, 0)),
            pl.BlockSpec((CE, 1), lambda bi, i: (0, 0)),
            pl.BlockSpec((2, 1, 1, T), lambda bi, i: (0, bi, 0, i)),
        ],
        out_specs=pl.BlockSpec((1, CE, T), lambda bi, i: (bi, 0, i)),
        out_shape=jax.ShapeDtypeStruct((Bn, CE, G), jnp.float32),
        compiler_params=pltpu.CompilerParams(
            dimension_semantics=("parallel", "parallel")),
    )(feat, W, b.reshape(CE, 1), wt)


def _sc_gather(svg, srg, linv, linr):
    PT = linv.shape[0]
    CH, CHL = _split(PT)
    mesh = plsc.VectorSubcoreMesh(core_axis_name="c", subcore_axis_name="s")

    @functools.partial(
        pl.kernel, mesh=mesh,
        out_type=[jax.ShapeDtypeStruct((PT,), jnp.float32),
                  jax.ShapeDtypeStruct((PT,), jnp.float32)],
        scratch_types=[pltpu.VMEM((CH,), jnp.int32),
                       pltpu.VMEM((CH,), jnp.int32),
                       pltpu.VMEM((CH,), jnp.float32),
                       pltpu.VMEM((CH,), jnp.float32),
                       pltpu.VMEM((CHL,), jnp.int32),
                       pltpu.VMEM((CHL,), jnp.int32),
                       pltpu.VMEM((CHL,), jnp.float32),
                       pltpu.VMEM((CHL,), jnp.float32),
                       pltpu.SemaphoreType.DMA,
                       pltpu.SemaphoreType.DMA],
    )
    def gk(svg_h, srg_h, lv_h, lr_h, ov_h, or_h,
           iv, ir, rv, rr, ivt, irt, rvt, rrt, sem1, sem2):
        wid = lax.axis_index("s") * _NC + lax.axis_index("c")
        base = wid * CH

        @pl.when(wid < _NW - 1)
        def _main():
            pltpu.sync_copy(lv_h.at[pl.ds(base, CH)], iv)
            pltpu.sync_copy(lr_h.at[pl.ds(base, CH)], ir)
            c1 = pltpu.async_copy(svg_h.at[iv], rv, sem1)
            c2 = pltpu.async_copy(srg_h.at[ir], rr, sem2)
            c1.wait()
            pltpu.sync_copy(rv, ov_h.at[pl.ds(base, CH)])
            c2.wait()
            pltpu.sync_copy(rr, or_h.at[pl.ds(base, CH)])

        @pl.when(wid == _NW - 1)
        def _tail():
            pltpu.sync_copy(lv_h.at[pl.ds(base, CHL)], ivt)
            pltpu.sync_copy(lr_h.at[pl.ds(base, CHL)], irt)
            c1 = pltpu.async_copy(svg_h.at[ivt], rvt, sem1)
            c2 = pltpu.async_copy(srg_h.at[irt], rrt, sem2)
            c1.wait()
            pltpu.sync_copy(rvt, ov_h.at[pl.ds(base, CHL)])
            c2.wait()
            pltpu.sync_copy(rrt, or_h.at[pl.ds(base, CHL)])

    return gk(svg, srg, linv, linr)


def _sc_scatter(avp, linv, arp, linr, zrow, GV, GR):
    PT = avp.shape[0]
    CH, CHL = _split(PT)
    SV = GV // _NS
    SR = GR // _NS
    mesh = plsc.VectorSubcoreMesh(core_axis_name="c", subcore_axis_name="s")

    @functools.partial(
        pl.kernel, mesh=mesh,
        out_type=[jax.ShapeDtypeStruct((_NC, GV), jnp.float32),
                  jax.ShapeDtypeStruct((_NC, GR), jnp.float32)],
        scratch_types=[pltpu.VMEM((CH,), jnp.int32),
                       pltpu.VMEM((CH,), jnp.int32),
                       pltpu.VMEM((CH,), jnp.float32),
                       pltpu.VMEM((CH,), jnp.float32),
                       pltpu.VMEM((CHL,), jnp.int32),
                       pltpu.VMEM((CHL,), jnp.int32),
                       pltpu.VMEM((CHL,), jnp.float32),
                       pltpu.VMEM((CHL,), jnp.float32),
                       pltpu.VMEM((SR,), jnp.float32),
                       pltpu.VMEM_SHARED((GV,), jnp.float32),
                       pltpu.VMEM_SHARED((GR,), jnp.float32),
                       pltpu.SemaphoreType.DMA],
    )
    def sk(av_h, lv_h, ar_h, lr_h, z_h, ov_h, or_h,
           iv, ir, vv, vr, ivt, irt, vvt, vrt, buf_v, accv, accr, sem):
        cid = lax.axis_index("c")
        sid = lax.axis_index("s")
        wid = sid * _NC + cid
        base = wid * CH
        pltpu.sync_copy(z_h.at[pl.ds(0, SV)], accv.at[pl.ds(sid * SV, SV)])
        pltpu.sync_copy(z_h, accr.at[pl.ds(sid * SR, SR)])
        plsc.subcore_barrier()

        @pl.when(wid < _NW - 1)
        def _main():
            pltpu.sync_copy(lv_h.at[pl.ds(base, CH)], iv)
            pltpu.sync_copy(av_h.at[pl.ds(base, CH)], vv)
            pltpu.sync_copy(lr_h.at[pl.ds(base, CH)], ir)
            pltpu.sync_copy(ar_h.at[pl.ds(base, CH)], vr)
            pltpu.sync_copy(vv, accv.at[iv], add=True)
            pltpu.sync_copy(vr, accr.at[ir], add=True)

        @pl.when(wid == _NW - 1)
        def _tail():
            pltpu.sync_copy(lv_h.at[pl.ds(base, CHL)], ivt)
            pltpu.sync_copy(av_h.at[pl.ds(base, CHL)], vvt)
            pltpu.sync_copy(lr_h.at[pl.ds(base, CHL)], irt)
            pltpu.sync_copy(ar_h.at[pl.ds(base, CHL)], vrt)
            pltpu.sync_copy(vvt, accv.at[ivt], add=True)
            pltpu.sync_copy(vrt, accr.at[irt], add=True)

        plsc.subcore_barrier()
        pltpu.sync_copy(accv.at[pl.ds(sid * SV, SV)], buf_v.at[pl.ds(0, SV)])
        pltpu.sync_copy(buf_v.at[pl.ds(0, SV)], ov_h.at[cid, pl.ds(sid * SV, SV)])
        pltpu.sync_copy(accr.at[pl.ds(sid * SR, SR)], buf_v)
        pltpu.sync_copy(buf_v, or_h.at[cid, pl.ds(sid * SR, SR)])

    return sk(avp, linv, arp, linr, zrow)


def kernel(p_feat, v_feat, r_feat, v2p_ind, r2p_ind,
           Wp, bp, Wv, bv, Wr, br, ap, av, ar):
    B, CP, N = p_feat.shape
    CV, HV, WVG = v_feat.shape[1:]
    CR, HR, WRG = r_feat.shape[1:]
    CE = Wp.shape[0]
    GV = HV * WVG
    GR = HR * WRG

    vf = v_feat.reshape(B, CV, GV)
    rf = r_feat.reshape(B, CR, GR)

    sv_grid = _score_grid(vf, Wv, av, bv, T=8192)
    sr_grid = _score_grid(rf, Wr, ar, br, T=8192)

    vi = v2p_ind.astype(jnp.int32)
    ri = r2p_ind.astype(jnp.int32)
    offv = (jnp.arange(B, dtype=jnp.int32) * GV)[:, None]
    offr = (jnp.arange(B, dtype=jnp.int32) * GR)[:, None]
    linv = (vi[..., 0] * WVG + vi[..., 1] + offv).reshape(-1)
    linr = (ri[..., 0] * WRG + ri[..., 1] + offr).reshape(-1)

    sv_f, sr_f = _sc_gather(sv_grid.reshape(-1), sr_grid.reshape(-1),
                            linv, linr)
    sv_pt = sv_f.reshape(B, 1, N)
    sr_pt = sr_f.reshape(B, 1, N)

    xp, alpha_v, alpha_r = _point_kernel(p_feat, sv_pt, sr_pt, Wp, bp, ap,
                                         T=2048)

    zrow = jnp.zeros((B * GR // _NS,), jnp.float32)
    wv2, wr2 = _sc_scatter(alpha_v.reshape(-1), linv, alpha_r.reshape(-1),
                           linr, zrow, B * GV, B * GR)

    xv = _map_kernel(vf, Wv, bv, wv2.reshape(_NC, B, 1, GV), T=8192)
    xr = _map_kernel(rf, Wr, br, wr2.reshape(_NC, B, 1, GR), T=8192)
    return (xp, xv.reshape(B, CE, HV, WVG), xr.reshape(B, CE, HR, WRG))


# speedup vs baseline: 34.1459x; 1.5086x over previous
import functools

import jax
import jax.numpy as jnp
from jax import lax
from jax.experimental import pallas as pl
from jax.experimental.pallas import tpu as pltpu
from jax.experimental.pallas import tpu_sc as plsc

_NC = 2
_NS = 16
_NW = _NC * _NS


def _split(total):
    ch = ((total + _NW - 1) // _NW + 7) // 8 * 8
    last = total - (_NW - 1) * ch
    assert last > 0 and last % 8 == 0 and ch % 8 == 0
    return ch, last


def _score_grid_body(f_ref, w_ref, a_ref, b_ref, s_ref):
    g = jnp.dot(a_ref[...], w_ref[...],
                preferred_element_type=jnp.float32)
    c0 = jnp.sum(a_ref[...] * b_ref[...])
    f = f_ref[0]
    s_ref[0, 0] = jnp.sum(f * g[0][:, None, None], axis=0) + c0


def _score_grid(feat, W, a, b, HB):
    Bn, C, H, Wg = feat.shape
    CE = W.shape[0]
    return pl.pallas_call(
        _score_grid_body,
        grid=(Bn, H // HB),
        in_specs=[
            pl.BlockSpec((1, C, HB, Wg), lambda bi, i: (bi, 0, i, 0)),
            pl.BlockSpec((CE, C), lambda bi, i: (0, 0)),
            pl.BlockSpec((1, CE), lambda bi, i: (0, 0)),
            pl.BlockSpec((1, CE), lambda bi, i: (0, 0)),
        ],
        out_specs=pl.BlockSpec((1, 1, HB, Wg), lambda bi, i: (bi, 0, i, 0)),
        out_shape=jax.ShapeDtypeStruct((Bn, 1, H, Wg), jnp.float32),
        compiler_params=pltpu.CompilerParams(
            dimension_semantics=("parallel", "parallel")),
    )(feat, W, a.reshape(1, CE), b.reshape(1, CE))


def _point_body(p_ref, sv_ref, sr_ref, w_ref, b_ref, a_ref,
                xp_ref, av_ref, ar_ref):
    emb = lax.dot_general(w_ref[...], p_ref[0],
                          (((1,), (1,)), ((), ())),
                          preferred_element_type=jnp.float32) + b_ref[...]
    sp = jnp.sum(emb * a_ref[...], axis=0, keepdims=True)
    sv = sv_ref[0]
    sr = sr_ref[0]
    m = jnp.maximum(sp, jnp.maximum(sv, sr))
    ep = jnp.exp(sp - m)
    ev = jnp.exp(sv - m)
    er = jnp.exp(sr - m)
    inv = 1.0 / (ep + ev + er)
    xp_ref[0] = emb * (ep * inv)
    av_ref[0] = ev * inv
    ar_ref[0] = er * inv


def _point_kernel(pT, sv_pt, sr_pt, Wp, bp, ap, T):
    Bn, N, C = pT.shape
    CE = Wp.shape[0]
    nt = pl.cdiv(N, T)
    return pl.pallas_call(
        _point_body,
        grid=(Bn, nt),
        in_specs=[
            pl.BlockSpec((1, T, C), lambda bi, i: (bi, i, 0)),
            pl.BlockSpec((1, 1, T), lambda bi, i: (bi, 0, i)),
            pl.BlockSpec((1, 1, T), lambda bi, i: (bi, 0, i)),
            pl.BlockSpec((CE, C), lambda bi, i: (0, 0)),
            pl.BlockSpec((CE, 1), lambda bi, i: (0, 0)),
            pl.BlockSpec((CE, 1), lambda bi, i: (0, 0)),
        ],
        out_specs=[
            pl.BlockSpec((1, CE, T), lambda bi, i: (bi, 0, i)),
            pl.BlockSpec((1, 1, T), lambda bi, i: (bi, 0, i)),
            pl.BlockSpec((1, 1, T), lambda bi, i: (bi, 0, i)),
        ],
        out_shape=[
            jax.ShapeDtypeStruct((Bn, CE, N), jnp.float32),
            jax.ShapeDtypeStruct((Bn, 1, N), jnp.float32),
            jax.ShapeDtypeStruct((Bn, 1, N), jnp.float32),
        ],
        compiler_params=pltpu.CompilerParams(
            dimension_semantics=("parallel", "parallel")),
    )(pT, sv_pt, sr_pt, Wp, bp.reshape(CE, 1), ap.reshape(CE, 1))


def _map_body(hb, f_ref, w_ref, b_ref, wt_ref, o_ref):
    for h in range(hb):
        emb = jnp.dot(w_ref[...], f_ref[0, :, h, :],
                      preferred_element_type=jnp.float32) + b_ref[...]
        wt = wt_ref[0, 0, h, :] + wt_ref[1, 0, h, :]
        o_ref[0, :, h, :] = emb * wt[None, :]


def _map_kernel(feat, W, b, wt, HB):
    Bn, C, H, Wg = feat.shape
    CE = W.shape[0]
    return pl.pallas_call(
        functools.partial(_map_body, HB),
        grid=(Bn, H // HB),
        in_specs=[
            pl.BlockSpec((1, C, HB, Wg), lambda bi, i: (bi, 0, i, 0)),
            pl.BlockSpec((CE, C), lambda bi, i: (0, 0)),
            pl.BlockSpec((CE, 1), lambda bi, i: (0, 0)),
            pl.BlockSpec((2, 1, HB, Wg), lambda bi, i: (0, bi, i, 0)),
        ],
        out_specs=pl.BlockSpec((1, CE, HB, Wg), lambda bi, i: (bi, 0, i, 0)),
        out_shape=jax.ShapeDtypeStruct((Bn, CE, H, Wg), jnp.float32),
        compiler_params=pltpu.CompilerParams(
            dimension_semantics=("parallel", "parallel")),
    )(feat, W, b.reshape(CE, 1), wt)


def _sc_gather(svg, srg, linv, linr):
    PT = linv.shape[0]
    CH, CHL = _split(PT)
    mesh = plsc.VectorSubcoreMesh(core_axis_name="c", subcore_axis_name="s")

    @functools.partial(
        pl.kernel, mesh=mesh,
        out_type=[jax.ShapeDtypeStruct((PT,), jnp.float32),
                  jax.ShapeDtypeStruct((PT,), jnp.float32)],
        scratch_types=[pltpu.VMEM((CH,), jnp.int32),
                       pltpu.VMEM((CH,), jnp.int32),
                       pltpu.VMEM((CH,), jnp.float32),
                       pltpu.VMEM((CH,), jnp.float32),
                       pltpu.VMEM((CHL,), jnp.int32),
                       pltpu.VMEM((CHL,), jnp.int32),
                       pltpu.VMEM((CHL,), jnp.float32),
                       pltpu.VMEM((CHL,), jnp.float32),
                       pltpu.SemaphoreType.DMA,
                       pltpu.SemaphoreType.DMA],
    )
    def gk(svg_h, srg_h, lv_h, lr_h, ov_h, or_h,
           iv, ir, rv, rr, ivt, irt, rvt, rrt, sem1, sem2):
        wid = lax.axis_index("s") * _NC + lax.axis_index("c")
        base = wid * CH

        @pl.when(wid < _NW - 1)
        def _main():
            pltpu.sync_copy(lv_h.at[pl.ds(base, CH)], iv)
            pltpu.sync_copy(lr_h.at[pl.ds(base, CH)], ir)
            c1 = pltpu.async_copy(svg_h.at[iv], rv, sem1)
            c2 = pltpu.async_copy(srg_h.at[ir], rr, sem2)
            c1.wait()
            pltpu.sync_copy(rv, ov_h.at[pl.ds(base, CH)])
            c2.wait()
            pltpu.sync_copy(rr, or_h.at[pl.ds(base, CH)])

        @pl.when(wid == _NW - 1)
        def _tail():
            pltpu.sync_copy(lv_h.at[pl.ds(base, CHL)], ivt)
            pltpu.sync_copy(lr_h.at[pl.ds(base, CHL)], irt)
            c1 = pltpu.async_copy(svg_h.at[ivt], rvt, sem1)
            c2 = pltpu.async_copy(srg_h.at[irt], rrt, sem2)
            c1.wait()
            pltpu.sync_copy(rvt, ov_h.at[pl.ds(base, CHL)])
            c2.wait()
            pltpu.sync_copy(rrt, or_h.at[pl.ds(base, CHL)])

    return gk(svg, srg, linv, linr)


def _sc_scatter(avp, linv, arp, linr, zrow, GV, GR):
    PT = avp.shape[0]
    CH, CHL = _split(PT)
    SV = GV // _NS
    SR = GR // _NS
    mesh = plsc.VectorSubcoreMesh(core_axis_name="c", subcore_axis_name="s")

    @functools.partial(
        pl.kernel, mesh=mesh,
        out_type=[jax.ShapeDtypeStruct((_NC, GV), jnp.float32),
                  jax.ShapeDtypeStruct((_NC, GR), jnp.float32)],
        scratch_types=[pltpu.VMEM((CH,), jnp.int32),
                       pltpu.VMEM((CH,), jnp.int32),
                       pltpu.VMEM((CH,), jnp.float32),
                       pltpu.VMEM((CH,), jnp.float32),
                       pltpu.VMEM((CHL,), jnp.int32),
                       pltpu.VMEM((CHL,), jnp.int32),
                       pltpu.VMEM((CHL,), jnp.float32),
                       pltpu.VMEM((CHL,), jnp.float32),
                       pltpu.VMEM((SR,), jnp.float32),
                       pltpu.VMEM_SHARED((GV,), jnp.float32),
                       pltpu.VMEM_SHARED((GR,), jnp.float32),
                       pltpu.SemaphoreType.DMA],
    )
    def sk(av_h, lv_h, ar_h, lr_h, z_h, ov_h, or_h,
           iv, ir, vv, vr, ivt, irt, vvt, vrt, buf_v, accv, accr, sem):
        cid = lax.axis_index("c")
        sid = lax.axis_index("s")
        wid = sid * _NC + cid
        base = wid * CH
        pltpu.sync_copy(z_h.at[pl.ds(0, SV)], accv.at[pl.ds(sid * SV, SV)])
        pltpu.sync_copy(z_h, accr.at[pl.ds(sid * SR, SR)])
        plsc.subcore_barrier()

        @pl.when(wid < _NW - 1)
        def _main():
            pltpu.sync_copy(lv_h.at[pl.ds(base, CH)], iv)
            pltpu.sync_copy(av_h.at[pl.ds(base, CH)], vv)
            pltpu.sync_copy(lr_h.at[pl.ds(base, CH)], ir)
            pltpu.sync_copy(ar_h.at[pl.ds(base, CH)], vr)
            pltpu.sync_copy(vv, accv.at[iv], add=True)
            pltpu.sync_copy(vr, accr.at[ir], add=True)

        @pl.when(wid == _NW - 1)
        def _tail():
            pltpu.sync_copy(lv_h.at[pl.ds(base, CHL)], ivt)
            pltpu.sync_copy(av_h.at[pl.ds(base, CHL)], vvt)
            pltpu.sync_copy(lr_h.at[pl.ds(base, CHL)], irt)
            pltpu.sync_copy(ar_h.at[pl.ds(base, CHL)], vrt)
            pltpu.sync_copy(vvt, accv.at[ivt], add=True)
            pltpu.sync_copy(vrt, accr.at[irt], add=True)

        plsc.subcore_barrier()
        pltpu.sync_copy(accv.at[pl.ds(sid * SV, SV)], buf_v.at[pl.ds(0, SV)])
        pltpu.sync_copy(buf_v.at[pl.ds(0, SV)], ov_h.at[cid, pl.ds(sid * SV, SV)])
        pltpu.sync_copy(accr.at[pl.ds(sid * SR, SR)], buf_v)
        pltpu.sync_copy(buf_v, or_h.at[cid, pl.ds(sid * SR, SR)])

    return sk(avp, linv, arp, linr, zrow)


def kernel(p_feat, v_feat, r_feat, v2p_ind, r2p_ind,
           Wp, bp, Wv, bv, Wr, br, ap, av, ar):
    B, CP, N = p_feat.shape
    CV, HV, WVG = v_feat.shape[1:]
    CR, HR, WRG = r_feat.shape[1:]
    CE = Wp.shape[0]
    GV = HV * WVG
    GR = HR * WRG

    sv_grid = _score_grid(v_feat, Wv, av, bv, HB=16)
    sr_grid = _score_grid(r_feat, Wr, ar, br, HB=8)

    vi = v2p_ind.astype(jnp.int32)
    ri = r2p_ind.astype(jnp.int32)
    offv = (jnp.arange(B, dtype=jnp.int32) * GV)[:, None]
    offr = (jnp.arange(B, dtype=jnp.int32) * GR)[:, None]
    linv = (vi[..., 0] * WVG + vi[..., 1] + offv).reshape(-1)
    linr = (ri[..., 0] * WRG + ri[..., 1] + offr).reshape(-1)

    sv_f, sr_f = _sc_gather(sv_grid.reshape(-1), sr_grid.reshape(-1),
                            linv, linr)
    sv_pt = sv_f.reshape(B, 1, N)
    sr_pt = sr_f.reshape(B, 1, N)

    pT = jnp.transpose(p_feat, (0, 2, 1))
    xp, alpha_v, alpha_r = _point_kernel(pT, sv_pt, sr_pt, Wp, bp, ap,
                                         T=2048)

    zrow = jnp.zeros((B * GR // _NS,), jnp.float32)
    wv2, wr2 = _sc_scatter(alpha_v.reshape(-1), linv, alpha_r.reshape(-1),
                           linr, zrow, B * GV, B * GR)

    xv = _map_kernel(v_feat, Wv, bv, wv2.reshape(_NC, B, HV, WVG), HB=16)
    xr = _map_kernel(r_feat, Wr, br, wr2.reshape(_NC, B, HR, WRG), HB=8)
    return (xp, xv, xr)


# speedup vs baseline: 36.1769x; 1.0595x over previous
import functools

import jax
import jax.numpy as jnp
from jax import lax
from jax.experimental import pallas as pl
from jax.experimental.pallas import tpu as pltpu
from jax.experimental.pallas import tpu_sc as plsc

_NC = 2
_NS = 16
_NW = _NC * _NS


def _split(total):
    ch = ((total + _NW - 1) // _NW + 7) // 8 * 8
    last = total - (_NW - 1) * ch
    assert last > 0 and last % 8 == 0 and ch % 8 == 0
    return ch, last


def _score_grid_body(f_ref, w_ref, a_ref, b_ref, s_ref):
    g = jnp.dot(a_ref[...], w_ref[...],
                preferred_element_type=jnp.float32)
    c0 = jnp.sum(a_ref[...] * b_ref[...])
    f = f_ref[0]
    s_ref[0, 0] = jnp.sum(f * g[0][:, None, None], axis=0) + c0


def _score_grid(feat, W, a, b, HB):
    Bn, C, H, Wg = feat.shape
    CE = W.shape[0]
    return pl.pallas_call(
        _score_grid_body,
        grid=(Bn, H // HB),
        in_specs=[
            pl.BlockSpec((1, C, HB, Wg), lambda bi, i: (bi, 0, i, 0)),
            pl.BlockSpec((CE, C), lambda bi, i: (0, 0)),
            pl.BlockSpec((1, CE), lambda bi, i: (0, 0)),
            pl.BlockSpec((1, CE), lambda bi, i: (0, 0)),
        ],
        out_specs=pl.BlockSpec((1, 1, HB, Wg), lambda bi, i: (bi, 0, i, 0)),
        out_shape=jax.ShapeDtypeStruct((Bn, 1, H, Wg), jnp.float32),
        compiler_params=pltpu.CompilerParams(
            dimension_semantics=("parallel", "parallel")),
    )(feat, W, a.reshape(1, CE), b.reshape(1, CE))


def _point_body(p_ref, sv_ref, sr_ref, w_ref, b_ref, a_ref,
                xp_ref, av_ref, ar_ref):
    emb = lax.dot_general(w_ref[...], p_ref[0],
                          (((1,), (1,)), ((), ())),
                          preferred_element_type=jnp.float32) + b_ref[...]
    sp = jnp.sum(emb * a_ref[...], axis=0, keepdims=True)
    sv = sv_ref[0]
    sr = sr_ref[0]
    m = jnp.maximum(sp, jnp.maximum(sv, sr))
    ep = jnp.exp(sp - m)
    ev = jnp.exp(sv - m)
    er = jnp.exp(sr - m)
    inv = 1.0 / (ep + ev + er)
    xp_ref[0] = emb * (ep * inv)
    av_ref[0] = ev * inv
    ar_ref[0] = er * inv


def _point_kernel(pT, sv_pt, sr_pt, Wp, bp, ap, T):
    Bn, N, C = pT.shape
    CE = Wp.shape[0]
    nt = pl.cdiv(N, T)
    return pl.pallas_call(
        _point_body,
        grid=(Bn, nt),
        in_specs=[
            pl.BlockSpec((1, T, C), lambda bi, i: (bi, i, 0)),
            pl.BlockSpec((1, 1, T), lambda bi, i: (bi, 0, i)),
            pl.BlockSpec((1, 1, T), lambda bi, i: (bi, 0, i)),
            pl.BlockSpec((CE, C), lambda bi, i: (0, 0)),
            pl.BlockSpec((CE, 1), lambda bi, i: (0, 0)),
            pl.BlockSpec((CE, 1), lambda bi, i: (0, 0)),
        ],
        out_specs=[
            pl.BlockSpec((1, CE, T), lambda bi, i: (bi, 0, i)),
            pl.BlockSpec((1, 1, T), lambda bi, i: (bi, 0, i)),
            pl.BlockSpec((1, 1, T), lambda bi, i: (bi, 0, i)),
        ],
        out_shape=[
            jax.ShapeDtypeStruct((Bn, CE, N), jnp.float32),
            jax.ShapeDtypeStruct((Bn, 1, N), jnp.float32),
            jax.ShapeDtypeStruct((Bn, 1, N), jnp.float32),
        ],
        compiler_params=pltpu.CompilerParams(
            dimension_semantics=("parallel", "parallel")),
    )(pT, sv_pt, sr_pt, Wp, bp.reshape(CE, 1), ap.reshape(CE, 1))


def _map_body(hb, f_ref, w_ref, b_ref, wt_ref, o_ref):
    for h in range(hb):
        emb = jnp.dot(w_ref[...], f_ref[0, :, h, :],
                      preferred_element_type=jnp.float32) + b_ref[...]
        wt = wt_ref[0, 0, h, :] + wt_ref[1, 0, h, :]
        o_ref[0, :, h, :] = emb * wt[None, :]


def _map_kernel(feat, W, b, wt, HB):
    Bn, C, H, Wg = feat.shape
    CE = W.shape[0]
    return pl.pallas_call(
        functools.partial(_map_body, HB),
        grid=(Bn, H // HB),
        in_specs=[
            pl.BlockSpec((1, C, HB, Wg), lambda bi, i: (bi, 0, i, 0)),
            pl.BlockSpec((CE, C), lambda bi, i: (0, 0)),
            pl.BlockSpec((CE, 1), lambda bi, i: (0, 0)),
            pl.BlockSpec((2, 1, HB, Wg), lambda bi, i: (0, bi, i, 0)),
        ],
        out_specs=pl.BlockSpec((1, CE, HB, Wg), lambda bi, i: (bi, 0, i, 0)),
        out_shape=jax.ShapeDtypeStruct((Bn, CE, H, Wg), jnp.float32),
        compiler_params=pltpu.CompilerParams(
            dimension_semantics=("parallel", "parallel")),
    )(feat, W, b.reshape(CE, 1), wt)


def _sc_gather(tab, lin):
    PT = lin.shape[0]
    CH, CHL = _split(PT)
    mesh = plsc.VectorSubcoreMesh(core_axis_name="c", subcore_axis_name="s")

    @functools.partial(
        pl.kernel, mesh=mesh,
        out_type=jax.ShapeDtypeStruct((PT,), jnp.float32),
        scratch_types=[pltpu.VMEM((CH,), jnp.int32),
                       pltpu.VMEM((CH,), jnp.float32),
                       pltpu.VMEM((CHL,), jnp.int32),
                       pltpu.VMEM((CHL,), jnp.float32),
                       pltpu.SemaphoreType.DMA],
    )
    def gk(tab_h, lin_h, out_h, iv, rv, ivt, rvt, sem):
        wid = lax.axis_index("s") * _NC + lax.axis_index("c")
        base = wid * CH

        @pl.when(wid < _NW - 1)
        def _main():
            pltpu.sync_copy(lin_h.at[pl.ds(base, CH)], iv)
            pltpu.async_copy(tab_h.at[iv], rv, sem).wait()
            pltpu.sync_copy(rv, out_h.at[pl.ds(base, CH)])

        @pl.when(wid == _NW - 1)
        def _tail():
            pltpu.sync_copy(lin_h.at[pl.ds(base, CHL)], ivt)
            pltpu.async_copy(tab_h.at[ivt], rvt, sem).wait()
            pltpu.sync_copy(rvt, out_h.at[pl.ds(base, CHL)])

    return gk(tab, lin)


def _sc_scatter(vals, lin, zrow, G):
    PT = vals.shape[0]
    CH, CHL = _split(PT)
    SG = G // _NS
    mesh = plsc.VectorSubcoreMesh(core_axis_name="c", subcore_axis_name="s")

    @functools.partial(
        pl.kernel, mesh=mesh,
        out_type=jax.ShapeDtypeStruct((_NC, G), jnp.float32),
        scratch_types=[pltpu.VMEM((CH,), jnp.int32),
                       pltpu.VMEM((CH,), jnp.float32),
                       pltpu.VMEM((CHL,), jnp.int32),
                       pltpu.VMEM((CHL,), jnp.float32),
                       pltpu.VMEM((SG,), jnp.float32),
                       pltpu.VMEM_SHARED((G,), jnp.float32),
                       pltpu.SemaphoreType.DMA],
    )
    def sk(v_h, l_h, z_h, o_h, iv, vv, ivt, vvt, buf_v, acc, sem):
        cid = lax.axis_index("c")
        sid = lax.axis_index("s")
        wid = sid * _NC + cid
        base = wid * CH
        pltpu.sync_copy(z_h.at[pl.ds(0, SG)], acc.at[pl.ds(sid * SG, SG)])
        plsc.subcore_barrier()

        @pl.when(wid < _NW - 1)
        def _main():
            pltpu.sync_copy(l_h.at[pl.ds(base, CH)], iv)
            pltpu.sync_copy(v_h.at[pl.ds(base, CH)], vv)
            pltpu.sync_copy(vv, acc.at[iv], add=True)

        @pl.when(wid == _NW - 1)
        def _tail():
            pltpu.sync_copy(l_h.at[pl.ds(base, CHL)], ivt)
            pltpu.sync_copy(v_h.at[pl.ds(base, CHL)], vvt)
            pltpu.sync_copy(vvt, acc.at[ivt], add=True)

        plsc.subcore_barrier()
        pltpu.sync_copy(acc.at[pl.ds(sid * SG, SG)], buf_v)
        pltpu.sync_copy(buf_v, o_h.at[cid, pl.ds(sid * SG, SG)])

    return sk(vals, lin, zrow)


def kernel(p_feat, v_feat, r_feat, v2p_ind, r2p_ind,
           Wp, bp, Wv, bv, Wr, br, ap, av, ar):
    B, CP, N = p_feat.shape
    CV, HV, WVG = v_feat.shape[1:]
    CR, HR, WRG = r_feat.shape[1:]
    CE = Wp.shape[0]
    GV = HV * WVG
    GR = HR * WRG

    sv_grid = _score_grid(v_feat, Wv, av, bv, HB=16)
    sr_grid = _score_grid(r_feat, Wr, ar, br, HB=8)

    vi = v2p_ind.astype(jnp.int32)
    ri = r2p_ind.astype(jnp.int32)
    offv = (jnp.arange(B, dtype=jnp.int32) * GV)[:, None]
    offr = (jnp.arange(B, dtype=jnp.int32) * GR)[:, None]
    linv = (vi[..., 0] * WVG + vi[..., 1] + offv).reshape(-1)
    linr = (ri[..., 0] * WRG + ri[..., 1] + offr).reshape(-1)

    sv_pt = _sc_gather(sv_grid.reshape(-1), linv).reshape(B, 1, N)
    sr_pt = _sc_gather(sr_grid.reshape(-1), linr).reshape(B, 1, N)

    pT = jnp.transpose(p_feat, (0, 2, 1))
    xp, alpha_v, alpha_r = _point_kernel(pT, sv_pt, sr_pt, Wp, bp, ap,
                                         T=2048)

    zrow = jnp.zeros((B * GR // _NS,), jnp.float32)
    wv2 = _sc_scatter(alpha_v.reshape(-1), linv, zrow, B * GV)
    wr2 = _sc_scatter(alpha_r.reshape(-1), linr, zrow, B * GR)

    xv = _map_kernel(v_feat, Wv, bv, wv2.reshape(_NC, B, HV, WVG), HB=16)
    xr = _map_kernel(r_feat, Wr, br, wr2.reshape(_NC, B, HR, WRG), HB=8)
    return (xp, xv, xr)


# speedup vs baseline: 40.2684x; 1.1131x over previous
import functools

import jax
import jax.numpy as jnp
from jax import lax
from jax.experimental import pallas as pl
from jax.experimental.pallas import tpu as pltpu
from jax.experimental.pallas import tpu_sc as plsc

_NC = 2
_NS = 16
_NW = _NC * _NS


def _split(total):
    ch = ((total + _NW - 1) // _NW + 7) // 8 * 8
    last = total - (_NW - 1) * ch
    assert last > 0 and last % 8 == 0 and ch % 8 == 0
    return ch, last


def _score_grid_body(f_ref, w_ref, a_ref, b_ref, s_ref):
    g = jnp.dot(a_ref[...], w_ref[...],
                preferred_element_type=jnp.float32)
    c0 = jnp.sum(a_ref[...] * b_ref[...])
    f = f_ref[0]
    s_ref[0, 0] = jnp.sum(f * g[0][:, None, None], axis=0) + c0


def _score_grid(feat, W, a, b, HB):
    Bn, C, H, Wg = feat.shape
    CE = W.shape[0]
    return pl.pallas_call(
        _score_grid_body,
        grid=(Bn, H // HB),
        in_specs=[
            pl.BlockSpec((1, C, HB, Wg), lambda bi, i: (bi, 0, i, 0)),
            pl.BlockSpec((CE, C), lambda bi, i: (0, 0)),
            pl.BlockSpec((1, CE), lambda bi, i: (0, 0)),
            pl.BlockSpec((1, CE), lambda bi, i: (0, 0)),
        ],
        out_specs=pl.BlockSpec((1, 1, HB, Wg), lambda bi, i: (bi, 0, i, 0)),
        out_shape=jax.ShapeDtypeStruct((Bn, 1, H, Wg), jnp.float32),
        compiler_params=pltpu.CompilerParams(
            dimension_semantics=("parallel", "parallel")),
    )(feat, W, a.reshape(1, CE), b.reshape(1, CE))


def _point_body(p_ref, sv_ref, sr_ref, w_ref, b_ref, a_ref,
                xp_ref, av_ref, ar_ref):
    emb = lax.dot_general(w_ref[...], p_ref[0],
                          (((1,), (1,)), ((), ())),
                          preferred_element_type=jnp.float32) + b_ref[...]
    sp = jnp.sum(emb * a_ref[...], axis=0, keepdims=True)
    sv = sv_ref[0]
    sr = sr_ref[0]
    m = jnp.maximum(sp, jnp.maximum(sv, sr))
    ep = jnp.exp(sp - m)
    ev = jnp.exp(sv - m)
    er = jnp.exp(sr - m)
    inv = 1.0 / (ep + ev + er)
    xp_ref[0] = emb * (ep * inv)
    av_ref[0] = ev * inv
    ar_ref[0] = er * inv


def _point_kernel(pT, sv_pt, sr_pt, Wp, bp, ap, T):
    Bn, N, C = pT.shape
    CE = Wp.shape[0]
    nt = pl.cdiv(N, T)
    return pl.pallas_call(
        _point_body,
        grid=(Bn, nt),
        in_specs=[
            pl.BlockSpec((1, T, C), lambda bi, i: (bi, i, 0)),
            pl.BlockSpec((1, 1, T), lambda bi, i: (bi, 0, i)),
            pl.BlockSpec((1, 1, T), lambda bi, i: (bi, 0, i)),
            pl.BlockSpec((CE, C), lambda bi, i: (0, 0)),
            pl.BlockSpec((CE, 1), lambda bi, i: (0, 0)),
            pl.BlockSpec((CE, 1), lambda bi, i: (0, 0)),
        ],
        out_specs=[
            pl.BlockSpec((1, CE, T), lambda bi, i: (bi, 0, i)),
            pl.BlockSpec((1, 1, T), lambda bi, i: (bi, 0, i)),
            pl.BlockSpec((1, 1, T), lambda bi, i: (bi, 0, i)),
        ],
        out_shape=[
            jax.ShapeDtypeStruct((Bn, CE, N), jnp.float32),
            jax.ShapeDtypeStruct((Bn, 1, N), jnp.float32),
            jax.ShapeDtypeStruct((Bn, 1, N), jnp.float32),
        ],
        compiler_params=pltpu.CompilerParams(
            dimension_semantics=("parallel", "parallel")),
    )(pT, sv_pt, sr_pt, Wp, bp.reshape(CE, 1), ap.reshape(CE, 1))


def _map_body(hb, f_ref, w_ref, b_ref, wt_ref, o_ref):
    for h in range(hb):
        emb = jnp.dot(w_ref[...], f_ref[0, :, h, :],
                      preferred_element_type=jnp.float32) + b_ref[...]
        wt = wt_ref[0, 0, h, :] + wt_ref[1, 0, h, :]
        o_ref[0, :, h, :] = emb * wt[None, :]


def _map_kernel(feat, W, b, wt, HB):
    Bn, C, H, Wg = feat.shape
    CE = W.shape[0]
    return pl.pallas_call(
        functools.partial(_map_body, HB),
        grid=(Bn, H // HB),
        in_specs=[
            pl.BlockSpec((1, C, HB, Wg), lambda bi, i: (bi, 0, i, 0)),
            pl.BlockSpec((CE, C), lambda bi, i: (0, 0)),
            pl.BlockSpec((CE, 1), lambda bi, i: (0, 0)),
            pl.BlockSpec((2, 1, HB, Wg), lambda bi, i: (0, bi, i, 0)),
        ],
        out_specs=pl.BlockSpec((1, CE, HB, Wg), lambda bi, i: (bi, 0, i, 0)),
        out_shape=jax.ShapeDtypeStruct((Bn, CE, H, Wg), jnp.float32),
        compiler_params=pltpu.CompilerParams(
            dimension_semantics=("parallel", "parallel")),
    )(feat, W, b.reshape(CE, 1), wt)


def _sc_gather(tab, lin):
    PT = lin.shape[0]
    CH, CHL = _split(PT)
    mesh = plsc.VectorSubcoreMesh(core_axis_name="c", subcore_axis_name="s")

    @functools.partial(
        pl.kernel, mesh=mesh,
        out_type=jax.ShapeDtypeStruct((PT,), jnp.float32),
        scratch_types=[pltpu.VMEM((CH,), jnp.int32),
                       pltpu.VMEM((CH,), jnp.float32),
                       pltpu.VMEM((CHL,), jnp.int32),
                       pltpu.VMEM((CHL,), jnp.float32),
                       pltpu.SemaphoreType.DMA],
    )
    def gk(tab_h, lin_h, out_h, iv, rv, ivt, rvt, sem):
        wid = lax.axis_index("s") * _NC + lax.axis_index("c")
        base = wid * CH

        @pl.when(wid < _NW - 1)
        def _main():
            pltpu.sync_copy(lin_h.at[pl.ds(base, CH)], iv)
            pltpu.async_copy(tab_h.at[iv], rv, sem).wait()
            pltpu.sync_copy(rv, out_h.at[pl.ds(base, CH)])

        @pl.when(wid == _NW - 1)
        def _tail():
            pltpu.sync_copy(lin_h.at[pl.ds(base, CHL)], ivt)
            pltpu.async_copy(tab_h.at[ivt], rvt, sem).wait()
            pltpu.sync_copy(rvt, out_h.at[pl.ds(base, CHL)])

    return gk(tab, lin)


def _sc_scatter(vals, lin, zrow, G):
    PT = vals.shape[0]
    CH, CHL = _split(PT)
    SG = G // _NS
    mesh = plsc.VectorSubcoreMesh(core_axis_name="c", subcore_axis_name="s")

    @functools.partial(
        pl.kernel, mesh=mesh,
        out_type=jax.ShapeDtypeStruct((_NC, G), jnp.float32),
        scratch_types=[pltpu.VMEM((CH,), jnp.int32),
                       pltpu.VMEM((CH,), jnp.float32),
                       pltpu.VMEM((CHL,), jnp.int32),
                       pltpu.VMEM((CHL,), jnp.float32),
                       pltpu.VMEM((SG,), jnp.float32),
                       pltpu.VMEM_SHARED((G,), jnp.float32),
                       pltpu.SemaphoreType.DMA],
    )
    def sk(v_h, l_h, z_h, o_h, iv, vv, ivt, vvt, buf_v, acc, sem):
        cid = lax.axis_index("c")
        sid = lax.axis_index("s")
        wid = sid * _NC + cid
        base = wid * CH
        pltpu.sync_copy(z_h.at[pl.ds(0, SG)], acc.at[pl.ds(sid * SG, SG)])
        plsc.subcore_barrier()

        @pl.when(wid < _NW - 1)
        def _main():
            pltpu.sync_copy(l_h.at[pl.ds(base, CH)], iv)
            pltpu.sync_copy(v_h.at[pl.ds(base, CH)], vv)
            pltpu.sync_copy(vv, acc.at[iv], add=True)

        @pl.when(wid == _NW - 1)
        def _tail():
            pltpu.sync_copy(l_h.at[pl.ds(base, CHL)], ivt)
            pltpu.sync_copy(v_h.at[pl.ds(base, CHL)], vvt)
            pltpu.sync_copy(vvt, acc.at[ivt], add=True)

        plsc.subcore_barrier()
        pltpu.sync_copy(acc.at[pl.ds(sid * SG, SG)], buf_v)
        pltpu.sync_copy(buf_v, o_h.at[cid, pl.ds(sid * SG, SG)])

    return sk(vals, lin, zrow)


def kernel(p_feat, v_feat, r_feat, v2p_ind, r2p_ind,
           Wp, bp, Wv, bv, Wr, br, ap, av, ar):
    B, CP, N = p_feat.shape
    CV, HV, WVG = v_feat.shape[1:]
    CR, HR, WRG = r_feat.shape[1:]
    CE = Wp.shape[0]
    GV = HV * WVG
    GR = HR * WRG

    sv_grid = _score_grid(v_feat, Wv, av, bv, HB=32)
    sr_grid = _score_grid(r_feat, Wr, ar, br, HB=8)

    vi = v2p_ind.astype(jnp.int32)
    ri = r2p_ind.astype(jnp.int32)
    offv = (jnp.arange(B, dtype=jnp.int32) * GV)[:, None]
    offr = (jnp.arange(B, dtype=jnp.int32) * GR)[:, None]
    linv = (vi[..., 0] * WVG + vi[..., 1] + offv).reshape(-1)
    linr = (ri[..., 0] * WRG + ri[..., 1] + offr).reshape(-1)

    sv_pt = _sc_gather(sv_grid.reshape(-1), linv).reshape(B, 1, N)
    sr_pt = _sc_gather(sr_grid.reshape(-1), linr).reshape(B, 1, N)

    pT = jnp.transpose(p_feat, (0, 2, 1))
    xp, alpha_v, alpha_r = _point_kernel(pT, sv_pt, sr_pt, Wp, bp, ap,
                                         T=4096)

    zrow = jnp.zeros((B * GR // _NS,), jnp.float32)
    wv2 = _sc_scatter(alpha_v.reshape(-1), linv, zrow, B * GV)

    xv = _map_kernel(v_feat, Wv, bv, wv2.reshape(_NC, B, HV, WVG), HB=16)
    wr2 = _sc_scatter(alpha_r.reshape(-1), linr, zrow, B * GR)
    xr = _map_kernel(r_feat, Wr, br, wr2.reshape(_NC, B, HR, WRG), HB=8)
    return (xp, xv, xr)


# speedup vs baseline: 44.2378x; 1.0986x over previous
import functools

import jax
import jax.numpy as jnp
from jax import lax
from jax.experimental import pallas as pl
from jax.experimental.pallas import tpu as pltpu
from jax.experimental.pallas import tpu_sc as plsc

_NC = 2
_NS = 16
_NW = _NC * _NS


def _split(total):
    ch = ((total + _NW - 1) // _NW + 7) // 8 * 8
    last = total - (_NW - 1) * ch
    assert last > 0 and last % 8 == 0 and ch % 8 == 0
    return ch, last


def _score_grid_body(f_ref, w_ref, a_ref, b_ref, s_ref):
    g = jnp.dot(a_ref[...], w_ref[...],
                preferred_element_type=jnp.float32)
    c0 = jnp.sum(a_ref[...] * b_ref[...])
    f = f_ref[0]
    s_ref[0, 0] = jnp.sum(f * g[0][:, None, None], axis=0) + c0


def _score_grid(feat, W, a, b, HB):
    Bn, C, H, Wg = feat.shape
    CE = W.shape[0]
    return pl.pallas_call(
        _score_grid_body,
        grid=(Bn, H // HB),
        in_specs=[
            pl.BlockSpec((1, C, HB, Wg), lambda bi, i: (bi, 0, i, 0)),
            pl.BlockSpec((CE, C), lambda bi, i: (0, 0)),
            pl.BlockSpec((1, CE), lambda bi, i: (0, 0)),
            pl.BlockSpec((1, CE), lambda bi, i: (0, 0)),
        ],
        out_specs=pl.BlockSpec((1, 1, HB, Wg), lambda bi, i: (bi, 0, i, 0)),
        out_shape=jax.ShapeDtypeStruct((Bn, 1, H, Wg), jnp.float32),
        compiler_params=pltpu.CompilerParams(
            dimension_semantics=("parallel", "parallel")),
    )(feat, W, a.reshape(1, CE), b.reshape(1, CE))


def _point_body(p_ref, sv_ref, sr_ref, w_ref, b_ref, a_ref,
                xp_ref, av_ref, ar_ref):
    emb = lax.dot_general(w_ref[...], p_ref[0],
                          (((1,), (1,)), ((), ())),
                          preferred_element_type=jnp.float32) + b_ref[...]
    sp = jnp.sum(emb * a_ref[...], axis=0, keepdims=True)
    sv = sv_ref[0]
    sr = sr_ref[0]
    m = jnp.maximum(sp, jnp.maximum(sv, sr))
    ep = jnp.exp(sp - m)
    ev = jnp.exp(sv - m)
    er = jnp.exp(sr - m)
    inv = 1.0 / (ep + ev + er)
    xp_ref[0] = emb * (ep * inv)
    av_ref[0] = ev * inv
    ar_ref[0] = er * inv


def _point_kernel(pT, sv_pt, sr_pt, Wp, bp, ap, T):
    Bn, N, C = pT.shape
    CE = Wp.shape[0]
    nt = pl.cdiv(N, T)
    return pl.pallas_call(
        _point_body,
        grid=(Bn, nt),
        in_specs=[
            pl.BlockSpec((1, T, C), lambda bi, i: (bi, i, 0)),
            pl.BlockSpec((1, 1, T), lambda bi, i: (bi, 0, i)),
            pl.BlockSpec((1, 1, T), lambda bi, i: (bi, 0, i)),
            pl.BlockSpec((CE, C), lambda bi, i: (0, 0)),
            pl.BlockSpec((CE, 1), lambda bi, i: (0, 0)),
            pl.BlockSpec((CE, 1), lambda bi, i: (0, 0)),
        ],
        out_specs=[
            pl.BlockSpec((1, CE, T), lambda bi, i: (bi, 0, i)),
            pl.BlockSpec((1, 1, T), lambda bi, i: (bi, 0, i)),
            pl.BlockSpec((1, 1, T), lambda bi, i: (bi, 0, i)),
        ],
        out_shape=[
            jax.ShapeDtypeStruct((Bn, CE, N), jnp.float32),
            jax.ShapeDtypeStruct((Bn, 1, N), jnp.float32),
            jax.ShapeDtypeStruct((Bn, 1, N), jnp.float32),
        ],
        compiler_params=pltpu.CompilerParams(
            dimension_semantics=("parallel", "parallel")),
    )(pT, sv_pt, sr_pt, Wp, bp.reshape(CE, 1), ap.reshape(CE, 1))


def _map_body(hb, f_ref, w_ref, b_ref, wt_ref, o_ref):
    for h in range(hb):
        emb = jnp.dot(w_ref[...], f_ref[0, :, h, :],
                      preferred_element_type=jnp.float32) + b_ref[...]
        wt = wt_ref[0, 0, h, :] + wt_ref[1, 0, h, :]
        o_ref[0, :, h, :] = emb * wt[None, :]


def _map_kernel(feat, W, b, wt, HB):
    Bn, C, H, Wg = feat.shape
    CE = W.shape[0]
    return pl.pallas_call(
        functools.partial(_map_body, HB),
        grid=(Bn, H // HB),
        in_specs=[
            pl.BlockSpec((1, C, HB, Wg), lambda bi, i: (bi, 0, i, 0)),
            pl.BlockSpec((CE, C), lambda bi, i: (0, 0)),
            pl.BlockSpec((CE, 1), lambda bi, i: (0, 0)),
            pl.BlockSpec((2, 1, HB, Wg), lambda bi, i: (0, bi, i, 0)),
        ],
        out_specs=pl.BlockSpec((1, CE, HB, Wg), lambda bi, i: (bi, 0, i, 0)),
        out_shape=jax.ShapeDtypeStruct((Bn, CE, H, Wg), jnp.float32),
        compiler_params=pltpu.CompilerParams(
            dimension_semantics=("parallel", "parallel")),
    )(feat, W, b.reshape(CE, 1), wt)


def _sc_gather(tab, lin):
    PT = lin.shape[0]
    CH, CHL = _split(PT)
    mesh = plsc.VectorSubcoreMesh(core_axis_name="c", subcore_axis_name="s")

    @functools.partial(
        pl.kernel, mesh=mesh,
        out_type=jax.ShapeDtypeStruct((PT,), jnp.float32),
        scratch_types=[pltpu.VMEM((CH,), jnp.int32),
                       pltpu.VMEM((CH,), jnp.float32),
                       pltpu.VMEM((CHL,), jnp.int32),
                       pltpu.VMEM((CHL,), jnp.float32),
                       pltpu.SemaphoreType.DMA],
    )
    def gk(tab_h, lin_h, out_h, iv, rv, ivt, rvt, sem):
        wid = lax.axis_index("s") * _NC + lax.axis_index("c")
        base = wid * CH

        @pl.when(wid < _NW - 1)
        def _main():
            pltpu.sync_copy(lin_h.at[pl.ds(base, CH)], iv)
            pltpu.async_copy(tab_h.at[iv], rv, sem).wait()
            pltpu.sync_copy(rv, out_h.at[pl.ds(base, CH)])

        @pl.when(wid == _NW - 1)
        def _tail():
            pltpu.sync_copy(lin_h.at[pl.ds(base, CHL)], ivt)
            pltpu.async_copy(tab_h.at[ivt], rvt, sem).wait()
            pltpu.sync_copy(rvt, out_h.at[pl.ds(base, CHL)])

    return gk(tab, lin)


def _sc_scatter(vals, lin, zrow, G):
    PT = vals.shape[0]
    CH, CHL = _split(PT)
    SG = G // _NS
    mesh = plsc.VectorSubcoreMesh(core_axis_name="c", subcore_axis_name="s")

    @functools.partial(
        pl.kernel, mesh=mesh,
        out_type=jax.ShapeDtypeStruct((_NC, G), jnp.float32),
        scratch_types=[pltpu.VMEM((CH,), jnp.int32),
                       pltpu.VMEM((CH,), jnp.float32),
                       pltpu.VMEM((CHL,), jnp.int32),
                       pltpu.VMEM((CHL,), jnp.float32),
                       pltpu.VMEM((SG,), jnp.float32),
                       pltpu.VMEM_SHARED((G,), jnp.float32),
                       pltpu.SemaphoreType.DMA],
    )
    def sk(v_h, l_h, z_h, o_h, iv, vv, ivt, vvt, buf_v, acc, sem):
        cid = lax.axis_index("c")
        sid = lax.axis_index("s")
        wid = sid * _NC + cid
        base = wid * CH
        pltpu.sync_copy(z_h.at[pl.ds(0, SG)], acc.at[pl.ds(sid * SG, SG)])
        plsc.subcore_barrier()

        @pl.when(wid < _NW - 1)
        def _main():
            pltpu.sync_copy(l_h.at[pl.ds(base, CH)], iv)
            pltpu.sync_copy(v_h.at[pl.ds(base, CH)], vv)
            pltpu.sync_copy(vv, acc.at[iv], add=True)

        @pl.when(wid == _NW - 1)
        def _tail():
            pltpu.sync_copy(l_h.at[pl.ds(base, CHL)], ivt)
            pltpu.sync_copy(v_h.at[pl.ds(base, CHL)], vvt)
            pltpu.sync_copy(vvt, acc.at[ivt], add=True)

        plsc.subcore_barrier()
        pltpu.sync_copy(acc.at[pl.ds(sid * SG, SG)], o_h.at[cid, pl.ds(sid * SG, SG)])

    return sk(vals, lin, zrow)


def kernel(p_feat, v_feat, r_feat, v2p_ind, r2p_ind,
           Wp, bp, Wv, bv, Wr, br, ap, av, ar):
    B, CP, N = p_feat.shape
    CV, HV, WVG = v_feat.shape[1:]
    CR, HR, WRG = r_feat.shape[1:]
    CE = Wp.shape[0]
    GV = HV * WVG
    GR = HR * WRG

    sv_grid = _score_grid(v_feat, Wv, av, bv, HB=64)
    sr_grid = _score_grid(r_feat, Wr, ar, br, HB=16)

    vi = v2p_ind.astype(jnp.int32)
    ri = r2p_ind.astype(jnp.int32)
    offv = (jnp.arange(B, dtype=jnp.int32) * GV)[:, None]
    offr = (jnp.arange(B, dtype=jnp.int32) * GR)[:, None]
    linv = (vi[..., 0] * WVG + vi[..., 1] + offv).reshape(-1)
    linr = (ri[..., 0] * WRG + ri[..., 1] + offr).reshape(-1)

    sv_pt = _sc_gather(sv_grid.reshape(-1), linv).reshape(B, 1, N)
    sr_pt = _sc_gather(sr_grid.reshape(-1), linr).reshape(B, 1, N)

    pT = jnp.transpose(p_feat, (0, 2, 1))
    xp, alpha_v, alpha_r = _point_kernel(pT, sv_pt, sr_pt, Wp, bp, ap,
                                         T=8192)

    zrow = jnp.zeros((B * GR // _NS,), jnp.float32)
    wv2 = _sc_scatter(alpha_v.reshape(-1), linv, zrow, B * GV)

    xv = _map_kernel(v_feat, Wv, bv, wv2.reshape(_NC, B, HV, WVG), HB=32)
    wr2 = _sc_scatter(alpha_r.reshape(-1), linr, zrow, B * GR)
    xr = _map_kernel(r_feat, Wr, br, wr2.reshape(_NC, B, HR, WRG), HB=16)
    return (xp, xv, xr)


# speedup vs baseline: 45.0173x; 1.0176x over previous
import functools

import jax
import jax.numpy as jnp
from jax import lax
from jax.experimental import pallas as pl
from jax.experimental.pallas import tpu as pltpu
from jax.experimental.pallas import tpu_sc as plsc

_NC = 2
_NS = 16
_NW = _NC * _NS


def _split(total, nw=_NW):
    ch = ((total + nw - 1) // nw + 7) // 8 * 8
    last = total - (nw - 1) * ch
    assert last > 0 and last % 8 == 0 and ch % 8 == 0
    return ch, last


def _score_grid_body(f_ref, w_ref, a_ref, b_ref, s_ref):
    g = jnp.dot(a_ref[...], w_ref[...],
                preferred_element_type=jnp.float32)
    c0 = jnp.sum(a_ref[...] * b_ref[...])
    f = f_ref[0]
    s_ref[0, 0] = jnp.sum(f * g[0][:, None, None], axis=0) + c0


def _score_grid(feat, W, a, b, HB):
    Bn, C, H, Wg = feat.shape
    CE = W.shape[0]
    return pl.pallas_call(
        _score_grid_body,
        grid=(Bn, H // HB),
        in_specs=[
            pl.BlockSpec((1, C, HB, Wg), lambda bi, i: (bi, 0, i, 0)),
            pl.BlockSpec((CE, C), lambda bi, i: (0, 0)),
            pl.BlockSpec((1, CE), lambda bi, i: (0, 0)),
            pl.BlockSpec((1, CE), lambda bi, i: (0, 0)),
        ],
        out_specs=pl.BlockSpec((1, 1, HB, Wg), lambda bi, i: (bi, 0, i, 0)),
        out_shape=jax.ShapeDtypeStruct((Bn, 1, H, Wg), jnp.float32),
        compiler_params=pltpu.CompilerParams(
            dimension_semantics=("parallel", "parallel")),
    )(feat, W, a.reshape(1, CE), b.reshape(1, CE))


def _point_body(p_ref, sv_ref, sr_ref, w_ref, b_ref, a_ref,
                xp_ref, av_ref, ar_ref):
    emb = lax.dot_general(w_ref[...], p_ref[0],
                          (((1,), (1,)), ((), ())),
                          preferred_element_type=jnp.float32) + b_ref[...]
    sp = jnp.sum(emb * a_ref[...], axis=0, keepdims=True)
    sv = sv_ref[0]
    sr = sr_ref[0]
    m = jnp.maximum(sp, jnp.maximum(sv, sr))
    ep = jnp.exp(sp - m)
    ev = jnp.exp(sv - m)
    er = jnp.exp(sr - m)
    inv = 1.0 / (ep + ev + er)
    xp_ref[0] = emb * (ep * inv)
    av_ref[0] = ev * inv
    ar_ref[0] = er * inv


def _point_kernel(pT, sv_pt, sr_pt, Wp, bp, ap, T):
    Bn, N, C = pT.shape
    CE = Wp.shape[0]
    nt = pl.cdiv(N, T)
    return pl.pallas_call(
        _point_body,
        grid=(Bn, nt),
        in_specs=[
            pl.BlockSpec((1, T, C), lambda bi, i: (bi, i, 0)),
            pl.BlockSpec((1, 1, T), lambda bi, i: (bi, 0, i)),
            pl.BlockSpec((1, 1, T), lambda bi, i: (bi, 0, i)),
            pl.BlockSpec((CE, C), lambda bi, i: (0, 0)),
            pl.BlockSpec((CE, 1), lambda bi, i: (0, 0)),
            pl.BlockSpec((CE, 1), lambda bi, i: (0, 0)),
        ],
        out_specs=[
            pl.BlockSpec((1, CE, T), lambda bi, i: (bi, 0, i)),
            pl.BlockSpec((1, 1, T), lambda bi, i: (bi, 0, i)),
            pl.BlockSpec((1, 1, T), lambda bi, i: (bi, 0, i)),
        ],
        out_shape=[
            jax.ShapeDtypeStruct((Bn, CE, N), jnp.float32),
            jax.ShapeDtypeStruct((Bn, 1, N), jnp.float32),
            jax.ShapeDtypeStruct((Bn, 1, N), jnp.float32),
        ],
        compiler_params=pltpu.CompilerParams(
            dimension_semantics=("parallel", "parallel")),
    )(pT, sv_pt, sr_pt, Wp, bp.reshape(CE, 1), ap.reshape(CE, 1))


def _map_body(hb, f_ref, w_ref, b_ref, wt_ref, o_ref):
    for h in range(hb):
        emb = jnp.dot(w_ref[...], f_ref[0, :, h, :],
                      preferred_element_type=jnp.float32) + b_ref[...]
        o_ref[0, :, h, :] = emb * wt_ref[0, h, :][None, :]


def _map_kernel(feat, W, b, wt, HB):
    Bn, C, H, Wg = feat.shape
    CE = W.shape[0]
    return pl.pallas_call(
        functools.partial(_map_body, HB),
        grid=(Bn, H // HB),
        in_specs=[
            pl.BlockSpec((1, C, HB, Wg), lambda bi, i: (bi, 0, i, 0)),
            pl.BlockSpec((CE, C), lambda bi, i: (0, 0)),
            pl.BlockSpec((CE, 1), lambda bi, i: (0, 0)),
            pl.BlockSpec((1, HB, Wg), lambda bi, i: (bi, i, 0)),
        ],
        out_specs=pl.BlockSpec((1, CE, HB, Wg), lambda bi, i: (bi, 0, i, 0)),
        out_shape=jax.ShapeDtypeStruct((Bn, CE, H, Wg), jnp.float32),
        compiler_params=pltpu.CompilerParams(
            dimension_semantics=("parallel", "parallel")),
    )(feat, W, b.reshape(CE, 1), wt)


def _sc_gather(tab, lin):
    PT = lin.shape[0]
    CH, CHL = _split(PT)
    mesh = plsc.VectorSubcoreMesh(core_axis_name="c", subcore_axis_name="s")

    @functools.partial(
        pl.kernel, mesh=mesh,
        out_type=jax.ShapeDtypeStruct((PT,), jnp.float32),
        scratch_types=[pltpu.VMEM((CH,), jnp.int32),
                       pltpu.VMEM((CH,), jnp.float32),
                       pltpu.VMEM((CHL,), jnp.int32),
                       pltpu.VMEM((CHL,), jnp.float32),
                       pltpu.SemaphoreType.DMA],
    )
    def gk(tab_h, lin_h, out_h, iv, rv, ivt, rvt, sem):
        wid = lax.axis_index("s") * _NC + lax.axis_index("c")
        base = wid * CH

        @pl.when(wid < _NW - 1)
        def _main():
            pltpu.sync_copy(lin_h.at[pl.ds(base, CH)], iv)
            pltpu.async_copy(tab_h.at[iv], rv, sem).wait()
            pltpu.sync_copy(rv, out_h.at[pl.ds(base, CH)])

        @pl.when(wid == _NW - 1)
        def _tail():
            pltpu.sync_copy(lin_h.at[pl.ds(base, CHL)], ivt)
            pltpu.async_copy(tab_h.at[ivt], rvt, sem).wait()
            pltpu.sync_copy(rvt, out_h.at[pl.ds(base, CHL)])

    return gk(tab, lin)


def _sc_scatter(avals, rvals, linv, linr, zrow, GVt, GRt):
    PT = avals.shape[0]
    CH, CHL = _split(PT, _NS)
    SGV = GVt // _NS
    SGR = GRt // _NS
    mesh = plsc.VectorSubcoreMesh(core_axis_name="c", subcore_axis_name="s")

    @functools.partial(
        pl.kernel, mesh=mesh,
        out_type=[jax.ShapeDtypeStruct((GVt,), jnp.float32),
                  jax.ShapeDtypeStruct((GRt,), jnp.float32)],
        scratch_types=[pltpu.VMEM((CH,), jnp.int32),
                       pltpu.VMEM((CH,), jnp.float32),
                       pltpu.VMEM((CHL,), jnp.int32),
                       pltpu.VMEM((CHL,), jnp.float32),
                       pltpu.VMEM_SHARED((GVt,), jnp.float32),
                       pltpu.VMEM_SHARED((GRt,), jnp.float32),
                       pltpu.SemaphoreType.DMA],
    )
    def sk(av_h, ar_h, lv_h, lr_h, z_h, ov_h, or_h,
           iv, vv, ivt, vvt, accv, accr, sem):
        cid = lax.axis_index("c")
        sid = lax.axis_index("s")
        base = sid * CH

        @pl.when(cid == 0)
        def _zv():
            pltpu.sync_copy(z_h.at[pl.ds(0, SGV)],
                            accv.at[pl.ds(sid * SGV, SGV)])

        @pl.when(cid == 1)
        def _zr():
            pltpu.sync_copy(z_h.at[pl.ds(0, SGR)],
                            accr.at[pl.ds(sid * SGR, SGR)])

        plsc.subcore_barrier()

        @pl.when((cid == 0) & (sid < _NS - 1))
        def _v_main():
            pltpu.sync_copy(lv_h.at[pl.ds(base, CH)], iv)
            pltpu.sync_copy(av_h.at[pl.ds(base, CH)], vv)
            pltpu.sync_copy(vv, accv.at[iv], add=True)

        @pl.when((cid == 0) & (sid == _NS - 1))
        def _v_tail():
            pltpu.sync_copy(lv_h.at[pl.ds(base, CHL)], ivt)
            pltpu.sync_copy(av_h.at[pl.ds(base, CHL)], vvt)
            pltpu.sync_copy(vvt, accv.at[ivt], add=True)

        @pl.when((cid == 1) & (sid < _NS - 1))
        def _r_main():
            pltpu.sync_copy(lr_h.at[pl.ds(base, CH)], iv)
            pltpu.sync_copy(ar_h.at[pl.ds(base, CH)], vv)
            pltpu.sync_copy(vv, accr.at[iv], add=True)

        @pl.when((cid == 1) & (sid == _NS - 1))
        def _r_tail():
            pltpu.sync_copy(lr_h.at[pl.ds(base, CHL)], ivt)
            pltpu.sync_copy(ar_h.at[pl.ds(base, CHL)], vvt)
            pltpu.sync_copy(vvt, accr.at[ivt], add=True)

        plsc.subcore_barrier()

        @pl.when(cid == 0)
        def _pv():
            pltpu.sync_copy(accv.at[pl.ds(sid * SGV, SGV)],
                            ov_h.at[pl.ds(sid * SGV, SGV)])

        @pl.when(cid == 1)
        def _pr():
            pltpu.sync_copy(accr.at[pl.ds(sid * SGR, SGR)],
                            or_h.at[pl.ds(sid * SGR, SGR)])

    return sk(avals, rvals, linv, linr, zrow)


def kernel(p_feat, v_feat, r_feat, v2p_ind, r2p_ind,
           Wp, bp, Wv, bv, Wr, br, ap, av, ar):
    B, CP, N = p_feat.shape
    CV, HV, WVG = v_feat.shape[1:]
    CR, HR, WRG = r_feat.shape[1:]
    CE = Wp.shape[0]
    GV = HV * WVG
    GR = HR * WRG

    sv_grid = _score_grid(v_feat, Wv, av, bv, HB=64)
    sr_grid = _score_grid(r_feat, Wr, ar, br, HB=16)

    viT = jnp.transpose(v2p_ind, (0, 2, 1)).astype(jnp.int32)
    riT = jnp.transpose(r2p_ind, (0, 2, 1)).astype(jnp.int32)
    offv = (jnp.arange(B, dtype=jnp.int32) * GV)[:, None]
    offr = (jnp.arange(B, dtype=jnp.int32) * GR)[:, None]
    linv = (viT[:, 0, :] * WVG + viT[:, 1, :] + offv).reshape(-1)
    linr = (riT[:, 0, :] * WRG + riT[:, 1, :] + offr).reshape(-1)

    sv_pt = _sc_gather(sv_grid.reshape(-1), linv).reshape(B, 1, N)
    sr_pt = _sc_gather(sr_grid.reshape(-1), linr).reshape(B, 1, N)

    pT = jnp.transpose(p_feat, (0, 2, 1))
    xp, alpha_v, alpha_r = _point_kernel(pT, sv_pt, sr_pt, Wp, bp, ap,
                                         T=8192)

    zrow = jnp.zeros((B * GR // _NS,), jnp.float32)
    wv, wr = _sc_scatter(alpha_v.reshape(-1), alpha_r.reshape(-1),
                         linv, linr, zrow, B * GV, B * GR)

    xv = _map_kernel(v_feat, Wv, bv, wv.reshape(B, HV, WVG), HB=64)
    xr = _map_kernel(r_feat, Wr, br, wr.reshape(B, HR, WRG), HB=16)
    return (xp, xv, xr)
